# baseline (speedup 1.0000x reference)
import functools

import jax
import jax.numpy as jnp
from jax.experimental import pallas as pl
from jax.experimental.pallas import tpu as pltpu

N = 10000
M = 5000
FEAT = 128
HID = 64
OUT = 16
NNZ = 320000

BR = 1000


def _lrelu(x, slope):
    return jnp.where(x > 0, x, slope * x)


def _mm_kernel(a_ref, b_ref, o_ref):
    o_ref[...] = jnp.dot(a_ref[...], b_ref[...],
                         preferred_element_type=jnp.float32)


def _mm(a, b, br=BR):
    r, k = a.shape
    _, c = b.shape
    return pl.pallas_call(
        _mm_kernel,
        grid=(r // br,),
        in_specs=[
            pl.BlockSpec((br, k), lambda i: (i, 0)),
            pl.BlockSpec((k, c), lambda i: (0, 0)),
        ],
        out_specs=pl.BlockSpec((br, c), lambda i: (i, 0)),
        out_shape=jax.ShapeDtypeStruct((r, c), jnp.float32),
    )(a, b)


def _pre_kernel(x_ref, w_ref, att_ref, xl_ref, a_ref):
    xl = jnp.dot(x_ref[...], w_ref[...], preferred_element_type=jnp.float32)
    xl_ref[...] = xl
    a_ref[...] = jnp.dot(xl, att_ref[...], preferred_element_type=jnp.float32)


def _pre(x, w, att_half, br=BR):
    r = x.shape[0]
    return pl.pallas_call(
        _pre_kernel,
        grid=(r // br,),
        in_specs=[
            pl.BlockSpec((br, FEAT), lambda i: (i, 0)),
            pl.BlockSpec((FEAT, FEAT), lambda i: (0, 0)),
            pl.BlockSpec((FEAT, 1), lambda i: (0, 0)),
        ],
        out_specs=[
            pl.BlockSpec((br, FEAT), lambda i: (i, 0)),
            pl.BlockSpec((br, 1), lambda i: (i, 0)),
        ],
        out_shape=[
            jax.ShapeDtypeStruct((r, FEAT), jnp.float32),
            jax.ShapeDtypeStruct((r, 1), jnp.float32),
        ],
    )(x, w, att_half.reshape(FEAT, 1))


def _stats_kernel(h_ref, o_ref, acc):
    i = pl.program_id(0)

    @pl.when(i == 0)
    def _():
        acc[...] = jnp.zeros_like(acc)

    h = h_ref[...]
    s1 = jnp.sum(h, axis=0, keepdims=True)
    s2 = jnp.sum(h * h, axis=0, keepdims=True)
    acc[...] += jnp.concatenate([s1, s2], axis=0)

    @pl.when(i == pl.num_programs(0) - 1)
    def _():
        o_ref[...] = acc[...]


def _stats(h):
    r = h.shape[0]
    return pl.pallas_call(
        _stats_kernel,
        grid=(r // BR,),
        in_specs=[pl.BlockSpec((BR, FEAT), lambda i: (i, 0))],
        out_specs=pl.BlockSpec((2, FEAT), lambda i: (0, 0)),
        out_shape=jax.ShapeDtypeStruct((2, FEAT), jnp.float32),
        scratch_shapes=[pltpu.VMEM((2, FEAT), jnp.float32)],
    )(h)


def _apply_kernel(h_ref, st_ref, gw_ref, gb_ref, ms_ref, wf_ref, bf_ref,
                  wn_ref, att_ref, h_out, outl_ref, xl2_ref, a2_ref):
    st = st_ref[...]
    mean = st[0:1, :] / N
    ms = ms_ref[...]
    var = st[1:2, :] / N - (2.0 * ms - ms * ms) * mean * mean
    h = h_ref[...] - ms * mean
    h = gw_ref[...] * h * jax.lax.rsqrt(var + 1e-5) + gb_ref[...]
    h = _lrelu(h, 0.01)
    h_out[...] = h
    outl_ref[...] = _lrelu(
        jnp.dot(h, wf_ref[...], preferred_element_type=jnp.float32)
        + bf_ref[...], 0.01)
    xl2 = jnp.dot(h, wn_ref[...], preferred_element_type=jnp.float32)
    xl2_ref[...] = xl2
    a2_ref[...] = jnp.dot(xl2, att_ref[...], preferred_element_type=jnp.float32)


def _apply(hraw, st, gw, gb, ms, wf, bf, wn, att_half):
    r = hraw.shape[0]
    return pl.pallas_call(
        _apply_kernel,
        grid=(r // BR,),
        in_specs=[
            pl.BlockSpec((BR, FEAT), lambda i: (i, 0)),
            pl.BlockSpec((2, FEAT), lambda i: (0, 0)),
            pl.BlockSpec((1, FEAT), lambda i: (0, 0)),
            pl.BlockSpec((1, FEAT), lambda i: (0, 0)),
            pl.BlockSpec((1, FEAT), lambda i: (0, 0)),
            pl.BlockSpec((FEAT, HID), lambda i: (0, 0)),
            pl.BlockSpec((1, HID), lambda i: (0, 0)),
            pl.BlockSpec((FEAT, FEAT), lambda i: (0, 0)),
            pl.BlockSpec((FEAT, 1), lambda i: (0, 0)),
        ],
        out_specs=[
            pl.BlockSpec((BR, FEAT), lambda i: (i, 0)),
            pl.BlockSpec((BR, HID), lambda i: (i, 0)),
            pl.BlockSpec((BR, FEAT), lambda i: (i, 0)),
            pl.BlockSpec((BR, 1), lambda i: (i, 0)),
        ],
        out_shape=[
            jax.ShapeDtypeStruct((r, FEAT), jnp.float32),
            jax.ShapeDtypeStruct((r, HID), jnp.float32),
            jax.ShapeDtypeStruct((r, FEAT), jnp.float32),
            jax.ShapeDtypeStruct((r, 1), jnp.float32),
        ],
    )(hraw, st, gw.reshape(1, FEAT), gb.reshape(1, FEAT),
      ms.reshape(1, FEAT), wf, bf.reshape(1, HID), wn,
      att_half.reshape(FEAT, 1))


_KBLK = 400


def _pool_kernel(out_ref, a1_ref, ba1_ref, a2_ref, ba2_ref, sc_ref, t_acc):
    k = pl.program_id(0)
    kb = pl.num_programs(0)

    part = jax.lax.dot_general(
        out_ref[...], a1_ref[...], (((0,), (0,)), ((), ())),
        preferred_element_type=jnp.float32)

    @pl.when(k == 0)
    def _():
        t_acc[...] = part

    @pl.when(k > 0)
    def _():
        t_acc[...] += part

    @pl.when(k == kb - 1)
    def _():
        tb = jnp.maximum(t_acc[...] + ba1_ref[...], 0.0)
        sc = jnp.dot(tb, a2_ref[...], preferred_element_type=jnp.float32)
        s = jax.nn.sigmoid(sc + ba2_ref[...])
        sc_ref[...] = s - jnp.mean(s)


def _pool(out, a1, ba1, a2, ba2):
    f2 = out.shape[1]
    return pl.pallas_call(
        _pool_kernel,
        grid=(N // _KBLK,),
        in_specs=[
            pl.BlockSpec((_KBLK, f2), lambda k: (k, 0)),
            pl.BlockSpec((_KBLK, N), lambda k: (k, 0)),
            pl.BlockSpec((1, N), lambda k: (0, 0)),
            pl.BlockSpec((N, 1), lambda k: (0, 0)),
            pl.BlockSpec((1, 1), lambda k: (0, 0)),
        ],
        out_specs=pl.BlockSpec((f2, 1), lambda k: (0, 0)),
        out_shape=jax.ShapeDtypeStruct((f2, 1), jnp.float32),
        scratch_shapes=[
            pltpu.VMEM((f2, N), jnp.float32),
        ],
    )(out, a1, ba1.reshape(1, N), a2, ba2.reshape(1, 1))


def _logits_kernel(out_ref, sc_ref, wc_ref, bc_ref, o_ref):
    o_ref[...] = jnp.dot(out_ref[...] * sc_ref[...], wc_ref[...],
                         preferred_element_type=jnp.float32) + bc_ref[...]


def _logits(out, sc_row, wc, bc):
    f2 = out.shape[1]
    return pl.pallas_call(
        _logits_kernel,
        grid=(N // BR,),
        in_specs=[
            pl.BlockSpec((BR, f2), lambda i: (i, 0)),
            pl.BlockSpec((1, f2), lambda i: (0, 0)),
            pl.BlockSpec((f2, OUT), lambda i: (0, 0)),
            pl.BlockSpec((1, OUT), lambda i: (0, 0)),
        ],
        out_specs=pl.BlockSpec((BR, OUT), lambda i: (i, 0)),
        out_shape=jax.ShapeDtypeStruct((N, OUT), jnp.float32),
    )(out, sc_row, wc, bc.reshape(1, OUT))


def _edge_stage(xl, src, he, a_src, b_he, d_inv, bv_inv):
    alpha = a_src[src] + b_he[he]
    alpha = _lrelu(alpha, 0.2)
    amax = jax.ops.segment_max(alpha, he, num_segments=M)
    amax = jnp.where(jnp.isfinite(amax), amax, 0.0)
    e = jnp.exp(alpha - amax[he])
    denom = jax.ops.segment_sum(e, he, num_segments=M)
    alpha = e / (denom[he] + 1e-16)
    hmsg = xl[src] * alpha[:, None]
    hout = jax.ops.segment_sum(hmsg, he, num_segments=M) * bv_inv[:, None]
    nmsg = hout[he] * alpha[:, None]
    out = jax.ops.segment_sum(nmsg, src, num_segments=N) * d_inv[:, None]
    return out


def kernel(x, edge_index, edge_attr, W1, att1, b1, gn1_w, gn1_b, gn1_ms,
           Wf1, bf1, W2, att2, b2, gn2_w, gn2_b, gn2_ms, Wf2, bf2,
           A1, bA1, A2, bA2, Wc, bc):
    src = edge_index[0]
    he = edge_index[1]

    ones = jnp.ones((NNZ,), jnp.float32)
    dcnt = jax.ops.segment_sum(ones, src, num_segments=N)
    bcnt = jax.ops.segment_sum(ones, he, num_segments=M)
    d_inv = jnp.where(dcnt > 0, 1.0 / dcnt, 0.0)
    bv_inv = jnp.where(bcnt > 0, 1.0 / bcnt, 0.0)

    xl1, a1 = _pre(x, W1, att1[:FEAT])
    eal1, b1he = _pre(edge_attr, W1, att1[FEAT:])
    eal2, b2he = _pre(edge_attr, W2, att2[FEAT:])
    a1 = a1[:, 0]
    b1he = b1he[:, 0]
    b2he = b2he[:, 0]

    h1raw = _edge_stage(xl1, src, he, a1, b1he, d_inv, bv_inv) + b1
    st1 = _stats(h1raw)
    _, out1, xl2, a2 = _apply(h1raw, st1, gn1_w, gn1_b, gn1_ms, Wf1, bf1,
                              W2, att2[:FEAT])
    a2 = a2[:, 0]

    h2raw = _edge_stage(xl2, src, he, a2, b2he, d_inv, bv_inv) + b2
    st2 = _stats(h2raw)
    _, out2, _, _ = _apply(h2raw, st2, gn2_w, gn2_b, gn2_ms, Wf2, bf2,
                           W2, att2[:FEAT])

    out = jnp.concatenate([x, out1, out2], axis=1)
    sc = _pool(out, A1, bA1, A2, bA2)
    sc_row = sc.reshape(1, 2 * FEAT)
    return _logits(out, sc_row, Wc, bc)


# speedup vs baseline: 15.1833x; 15.1833x over previous
import functools

import jax
import jax.numpy as jnp
from jax import lax
from jax.experimental import pallas as pl
from jax.experimental.pallas import tpu as pltpu
from jax.experimental.pallas import tpu_sc as plsc

N = 10000
M = 5000
FEAT = 128
HID = 64
OUT = 16
NNZ = 320000

BR = 1000

NC = 2
NS = 16
NW = NC * NS
EPT = NNZ // NW
K_E = 80
NB = EPT // K_E
MP = 5008


def _lrelu(x, slope):
    return jnp.where(x > 0, x, slope * x)


def _mm_kernel(a_ref, b_ref, o_ref):
    o_ref[...] = jnp.dot(a_ref[...], b_ref[...],
                         preferred_element_type=jnp.float32)


def _mm(a, b, br=BR):
    r, k = a.shape
    _, c = b.shape
    return pl.pallas_call(
        _mm_kernel,
        grid=(r // br,),
        in_specs=[
            pl.BlockSpec((br, k), lambda i: (i, 0)),
            pl.BlockSpec((k, c), lambda i: (0, 0)),
        ],
        out_specs=pl.BlockSpec((br, c), lambda i: (i, 0)),
        out_shape=jax.ShapeDtypeStruct((r, c), jnp.float32),
    )(a, b)


def _pre_kernel(x_ref, w_ref, att_ref, xl_ref, a_ref):
    xl = jnp.dot(x_ref[...], w_ref[...], preferred_element_type=jnp.float32)
    xl_ref[...] = xl
    a_ref[...] = jnp.dot(xl, att_ref[...], preferred_element_type=jnp.float32)


def _pre(x, w, att_half, br=BR):
    r = x.shape[0]
    return pl.pallas_call(
        _pre_kernel,
        grid=(r // br,),
        in_specs=[
            pl.BlockSpec((br, FEAT), lambda i: (i, 0)),
            pl.BlockSpec((FEAT, FEAT), lambda i: (0, 0)),
            pl.BlockSpec((FEAT, 1), lambda i: (0, 0)),
        ],
        out_specs=[
            pl.BlockSpec((br, FEAT), lambda i: (i, 0)),
            pl.BlockSpec((br, 1), lambda i: (i, 0)),
        ],
        out_shape=[
            jax.ShapeDtypeStruct((r, FEAT), jnp.float32),
            jax.ShapeDtypeStruct((r, 1), jnp.float32),
        ],
    )(x, w, att_half.reshape(FEAT, 1))


def _stats_kernel(h_ref, o_ref, acc):
    i = pl.program_id(0)

    @pl.when(i == 0)
    def _():
        acc[...] = jnp.zeros_like(acc)

    h = h_ref[...]
    s1 = jnp.sum(h, axis=0, keepdims=True)
    s2 = jnp.sum(h * h, axis=0, keepdims=True)
    acc[...] += jnp.concatenate([s1, s2], axis=0)

    @pl.when(i == pl.num_programs(0) - 1)
    def _():
        o_ref[...] = acc[...]


def _stats(h):
    r = h.shape[0]
    return pl.pallas_call(
        _stats_kernel,
        grid=(r // BR,),
        in_specs=[pl.BlockSpec((BR, FEAT), lambda i: (i, 0))],
        out_specs=pl.BlockSpec((2, FEAT), lambda i: (0, 0)),
        out_shape=jax.ShapeDtypeStruct((2, FEAT), jnp.float32),
        scratch_shapes=[pltpu.VMEM((2, FEAT), jnp.float32)],
    )(h)


def _apply_kernel(h_ref, st_ref, gw_ref, gb_ref, ms_ref, wf_ref, bf_ref,
                  wn_ref, att_ref, h_out, outl_ref, xl2_ref, a2_ref):
    st = st_ref[...]
    mean = st[0:1, :] / N
    ms = ms_ref[...]
    var = st[1:2, :] / N - (2.0 * ms - ms * ms) * mean * mean
    h = h_ref[...] - ms * mean
    h = gw_ref[...] * h * jax.lax.rsqrt(var + 1e-5) + gb_ref[...]
    h = _lrelu(h, 0.01)
    h_out[...] = h
    outl_ref[...] = _lrelu(
        jnp.dot(h, wf_ref[...], preferred_element_type=jnp.float32)
        + bf_ref[...], 0.01)
    xl2 = jnp.dot(h, wn_ref[...], preferred_element_type=jnp.float32)
    xl2_ref[...] = xl2
    a2_ref[...] = jnp.dot(xl2, att_ref[...], preferred_element_type=jnp.float32)


def _apply(hraw, st, gw, gb, ms, wf, bf, wn, att_half):
    r = hraw.shape[0]
    return pl.pallas_call(
        _apply_kernel,
        grid=(r // BR,),
        in_specs=[
            pl.BlockSpec((BR, FEAT), lambda i: (i, 0)),
            pl.BlockSpec((2, FEAT), lambda i: (0, 0)),
            pl.BlockSpec((1, FEAT), lambda i: (0, 0)),
            pl.BlockSpec((1, FEAT), lambda i: (0, 0)),
            pl.BlockSpec((1, FEAT), lambda i: (0, 0)),
            pl.BlockSpec((FEAT, HID), lambda i: (0, 0)),
            pl.BlockSpec((1, HID), lambda i: (0, 0)),
            pl.BlockSpec((FEAT, FEAT), lambda i: (0, 0)),
            pl.BlockSpec((FEAT, 1), lambda i: (0, 0)),
        ],
        out_specs=[
            pl.BlockSpec((BR, FEAT), lambda i: (i, 0)),
            pl.BlockSpec((BR, HID), lambda i: (i, 0)),
            pl.BlockSpec((BR, FEAT), lambda i: (i, 0)),
            pl.BlockSpec((BR, 1), lambda i: (i, 0)),
        ],
        out_shape=[
            jax.ShapeDtypeStruct((r, FEAT), jnp.float32),
            jax.ShapeDtypeStruct((r, HID), jnp.float32),
            jax.ShapeDtypeStruct((r, FEAT), jnp.float32),
            jax.ShapeDtypeStruct((r, 1), jnp.float32),
        ],
    )(hraw, st, gw.reshape(1, FEAT), gb.reshape(1, FEAT),
      ms.reshape(1, FEAT), wf, bf.reshape(1, HID), wn,
      att_half.reshape(FEAT, 1))


_KBLK = 400


def _pool_kernel(out_ref, a1_ref, ba1_ref, a2_ref, ba2_ref, sc_ref, t_acc):
    k = pl.program_id(0)
    kb = pl.num_programs(0)

    part = jax.lax.dot_general(
        out_ref[...], a1_ref[...], (((0,), (0,)), ((), ())),
        preferred_element_type=jnp.float32)

    @pl.when(k == 0)
    def _():
        t_acc[...] = part

    @pl.when(k > 0)
    def _():
        t_acc[...] += part

    @pl.when(k == kb - 1)
    def _():
        tb = jnp.maximum(t_acc[...] + ba1_ref[...], 0.0)
        sc = jnp.dot(tb, a2_ref[...], preferred_element_type=jnp.float32)
        s = jax.nn.sigmoid(sc + ba2_ref[...])
        sc_ref[...] = s - jnp.mean(s)


def _pool(out, a1, ba1, a2, ba2):
    f2 = out.shape[1]
    return pl.pallas_call(
        _pool_kernel,
        grid=(N // _KBLK,),
        in_specs=[
            pl.BlockSpec((_KBLK, f2), lambda k: (k, 0)),
            pl.BlockSpec((_KBLK, N), lambda k: (k, 0)),
            pl.BlockSpec((1, N), lambda k: (0, 0)),
            pl.BlockSpec((N, 1), lambda k: (0, 0)),
            pl.BlockSpec((1, 1), lambda k: (0, 0)),
        ],
        out_specs=pl.BlockSpec((f2, 1), lambda k: (0, 0)),
        out_shape=jax.ShapeDtypeStruct((f2, 1), jnp.float32),
        scratch_shapes=[
            pltpu.VMEM((f2, N), jnp.float32),
        ],
    )(out, a1, ba1.reshape(1, N), a2, ba2.reshape(1, 1))


def _logits_kernel(out_ref, sc_ref, wc_ref, bc_ref, o_ref):
    o_ref[...] = jnp.dot(out_ref[...] * sc_ref[...], wc_ref[...],
                         preferred_element_type=jnp.float32) + bc_ref[...]


def _logits(out, sc_row, wc, bc):
    f2 = out.shape[1]
    return pl.pallas_call(
        _logits_kernel,
        grid=(N // BR,),
        in_specs=[
            pl.BlockSpec((BR, f2), lambda i: (i, 0)),
            pl.BlockSpec((1, f2), lambda i: (0, 0)),
            pl.BlockSpec((f2, OUT), lambda i: (0, 0)),
            pl.BlockSpec((1, OUT), lambda i: (0, 0)),
        ],
        out_specs=pl.BlockSpec((BR, OUT), lambda i: (i, 0)),
        out_shape=jax.ShapeDtypeStruct((N, OUT), jnp.float32),
    )(out, sc_row, wc, bc.reshape(1, OUT))


_MESH = plsc.VectorSubcoreMesh(core_axis_name="c", subcore_axis_name="s",
                               num_cores=NC, num_subcores=NS)
_SC_PARAMS = pltpu.CompilerParams(needs_layout_passes=False,
                                  use_tc_tiling_on_sc=False)


def _wid():
    return lax.axis_index("s") * NC + lax.axis_index("c")


def _sc_count_body(src_hbm, he_hbm, cn_hbm, cm_hbm, srcv, hev, cn, cm):
    wid = _wid()
    base = wid * EPT
    pltpu.sync_copy(src_hbm.at[pl.ds(base, EPT)], srcv)
    pltpu.sync_copy(he_hbm.at[pl.ds(base, EPT)], hev)
    zero16 = jnp.zeros((16,), jnp.float32)

    def zn(i, carry):
        cn[pl.ds(i * 16, 16)] = zero16
        return carry

    lax.fori_loop(0, N // 16, zn, 0)

    def zm(i, carry):
        cm[pl.ds(i * 16, 16)] = zero16
        return carry

    lax.fori_loop(0, MP // 16, zm, 0)
    ones16 = jnp.ones((16,), jnp.float32)

    def body(i, carry):
        sv = srcv[pl.ds(i * 16, 16)]
        hv = hev[pl.ds(i * 16, 16)]
        plsc.addupdate_scatter(cn, [sv], ones16)
        plsc.addupdate_scatter(cm, [hv], ones16)
        return carry

    lax.fori_loop(0, EPT // 16, body, 0)
    pltpu.sync_copy(cn, cn_hbm.at[wid])
    pltpu.sync_copy(cm, cm_hbm.at[wid])


_sc_count = pl.kernel(
    _sc_count_body,
    out_type=[
        jax.ShapeDtypeStruct((NW, N), jnp.float32),
        jax.ShapeDtypeStruct((NW, MP), jnp.float32),
    ],
    mesh=_MESH,
    compiler_params=_SC_PARAMS,
    scratch_types=[
        pltpu.VMEM((EPT,), jnp.int32),
        pltpu.VMEM((EPT,), jnp.int32),
        pltpu.VMEM((N,), jnp.float32),
        pltpu.VMEM((MP,), jnp.float32),
    ],
)


def _sc_alpha_body(src_hbm, he_hbm, a_hbm, b_hbm, e_hbm, dp_hbm,
                   srcv, hev, av, bv, ev, dv):
    wid = _wid()
    base = wid * EPT
    pltpu.sync_copy(src_hbm.at[pl.ds(base, EPT)], srcv)
    pltpu.sync_copy(he_hbm.at[pl.ds(base, EPT)], hev)
    pltpu.sync_copy(a_hbm, av)
    pltpu.sync_copy(b_hbm, bv)
    zero16 = jnp.zeros((16,), jnp.float32)

    def zm(i, carry):
        dv[pl.ds(i * 16, 16)] = zero16
        return carry

    lax.fori_loop(0, MP // 16, zm, 0)

    def body(i, carry):
        sv = srcv[pl.ds(i * 16, 16)]
        hv = hev[pl.ds(i * 16, 16)]
        aa = plsc.load_gather(av, [sv])
        bb = plsc.load_gather(bv, [hv])
        z = aa + bb
        z = jnp.where(z > 0, z, 0.2 * z)
        e = jnp.exp(z)
        ev[pl.ds(i * 16, 16)] = e
        plsc.addupdate_scatter(dv, [hv], e)
        return carry

    lax.fori_loop(0, EPT // 16, body, 0)
    pltpu.sync_copy(ev, e_hbm.at[pl.ds(base, EPT)])
    pltpu.sync_copy(dv, dp_hbm.at[wid])


_sc_alpha = pl.kernel(
    _sc_alpha_body,
    out_type=[
        jax.ShapeDtypeStruct((NNZ,), jnp.float32),
        jax.ShapeDtypeStruct((NW, MP), jnp.float32),
    ],
    mesh=_MESH,
    compiler_params=_SC_PARAMS,
    scratch_types=[
        pltpu.VMEM((EPT,), jnp.int32),
        pltpu.VMEM((EPT,), jnp.int32),
        pltpu.VMEM((N,), jnp.float32),
        pltpu.VMEM((MP,), jnp.float32),
        pltpu.VMEM((EPT,), jnp.float32),
        pltpu.VMEM((MP,), jnp.float32),
    ],
)


def _sc_alpha2_body(he_hbm, e_hbm, dinv_hbm, al_hbm, hev, ev, dinv_v):
    wid = _wid()
    base = wid * EPT
    pltpu.sync_copy(he_hbm.at[pl.ds(base, EPT)], hev)
    pltpu.sync_copy(e_hbm.at[pl.ds(base, EPT)], ev)
    pltpu.sync_copy(dinv_hbm, dinv_v)

    def al(i, carry):
        hv = hev[pl.ds(i * 16, 16)]
        dd = plsc.load_gather(dinv_v, [hv])
        ev[pl.ds(i * 16, 16)] = ev[pl.ds(i * 16, 16)] * dd
        return carry

    lax.fori_loop(0, EPT // 16, al, 0)
    pltpu.sync_copy(ev, al_hbm.at[pl.ds(base, EPT)])


_sc_alpha2 = pl.kernel(
    _sc_alpha2_body,
    out_type=jax.ShapeDtypeStruct((NNZ,), jnp.float32),
    mesh=_MESH,
    compiler_params=_SC_PARAMS,
    scratch_types=[
        pltpu.VMEM((EPT,), jnp.int32),
        pltpu.VMEM((EPT,), jnp.float32),
        pltpu.VMEM((MP,), jnp.float32),
    ],
)


def _sc_msg_body(table_hbm, gidx_hbm, sidx_hbm, al_hbm, z_hbm, outp_hbm,
                 gidx_v, sidx_v, alv, buf, sh, sem, *, o_rows):
    c = lax.axis_index("c")
    s = lax.axis_index("s")
    wid = s * NC + c
    base = wid * EPT
    zr = o_rows // NS
    pltpu.sync_copy(gidx_hbm.at[wid], gidx_v)
    pltpu.sync_copy(sidx_hbm.at[wid], sidx_v)
    pltpu.sync_copy(al_hbm.at[pl.ds(base, EPT)], alv)
    pltpu.sync_copy(z_hbm, sh.at[pl.ds(s * zr, zr)])
    plsc.subcore_barrier()

    def batch(j, carry):
        pltpu.async_copy(table_hbm.at[gidx_v.at[j]], buf, sem).wait()
        jb = j * K_E
        for ii in range(K_E // 16):
            av16 = alv[pl.ds(jb + ii * 16, 16)]
            for l in range(16):
                a = av16[l]
                row = ii * 16 + l
                for d in range(8):
                    buf[row, pl.ds(d * 16, 16)] = (
                        buf[row, pl.ds(d * 16, 16)] * a)
        pltpu.sync_copy(buf, sh.at[sidx_v.at[j]], add=True)
        return carry

    lax.fori_loop(0, NB, batch, 0)
    plsc.subcore_barrier()
    pltpu.sync_copy(sh.at[pl.ds(s * zr, zr)],
                    outp_hbm.at[c, pl.ds(s * zr, zr)])


def _make_sc_msg(t_rows, o_rows):
    return pl.kernel(
        functools.partial(_sc_msg_body, o_rows=o_rows),
        out_type=jax.ShapeDtypeStruct((NC, o_rows, FEAT), jnp.float32),
        mesh=_MESH,
        compiler_params=_SC_PARAMS,
        scratch_types=[
            pltpu.VMEM((NB, K_E), jnp.int32),
            pltpu.VMEM((NB, K_E), jnp.int32),
            pltpu.VMEM((EPT,), jnp.float32),
            pltpu.VMEM((K_E, FEAT), jnp.float32),
            pltpu.VMEM_SHARED((o_rows, FEAT), jnp.float32),
            pltpu.SemaphoreType.DMA,
        ],
    )


_sc_msg_he = _make_sc_msg(N, MP)
_sc_msg_node = _make_sc_msg(MP, N)


def _colsum_kernel(p_ref, o_ref, *, mode):
    ssum = jnp.sum(p_ref[...], axis=0, keepdims=True)
    if mode == "rcp":
        o_ref[...] = 1.0 / (ssum + 1e-16)
    else:
        o_ref[...] = jnp.where(ssum > 0, 1.0 / ssum, 0.0)


def _colsum(p, mode):
    nw, c = p.shape
    return pl.pallas_call(
        functools.partial(_colsum_kernel, mode=mode),
        grid=(1,),
        in_specs=[pl.BlockSpec((nw, c), lambda i: (0, 0))],
        out_specs=pl.BlockSpec((1, c), lambda i: (0, 0)),
        out_shape=jax.ShapeDtypeStruct((1, c), jnp.float32),
    )(p)


def _chout_kernel(p_ref, bv_ref, o_ref):
    o_ref[...] = (p_ref[0] + p_ref[1]) * bv_ref[...]


def _chout(p, bv_col):
    r = p.shape[1]
    return pl.pallas_call(
        _chout_kernel,
        grid=(1,),
        in_specs=[
            pl.BlockSpec((NC, r, FEAT), lambda i: (0, 0, 0)),
            pl.BlockSpec((r, 1), lambda i: (0, 0)),
        ],
        out_specs=pl.BlockSpec((r, FEAT), lambda i: (0, 0)),
        out_shape=jax.ShapeDtypeStruct((r, FEAT), jnp.float32),
    )(p, bv_col)


def _chraw_kernel(p_ref, dinv_ref, b_ref, h_ref, st_ref, acc):
    i = pl.program_id(0)

    @pl.when(i == 0)
    def _():
        acc[...] = jnp.zeros_like(acc)

    h = (p_ref[0] + p_ref[1]) * dinv_ref[...] + b_ref[...]
    h_ref[...] = h
    s1 = jnp.sum(h, axis=0, keepdims=True)
    s2 = jnp.sum(h * h, axis=0, keepdims=True)
    acc[...] += jnp.concatenate([s1, s2], axis=0)

    @pl.when(i == pl.num_programs(0) - 1)
    def _():
        st_ref[...] = acc[...]


def _chraw(p, dinv_col, b):
    return pl.pallas_call(
        _chraw_kernel,
        grid=(N // BR,),
        in_specs=[
            pl.BlockSpec((NC, BR, FEAT), lambda i: (0, i, 0)),
            pl.BlockSpec((BR, 1), lambda i: (i, 0)),
            pl.BlockSpec((1, FEAT), lambda i: (0, 0)),
        ],
        out_specs=[
            pl.BlockSpec((BR, FEAT), lambda i: (i, 0)),
            pl.BlockSpec((2, FEAT), lambda i: (0, 0)),
        ],
        out_shape=[
            jax.ShapeDtypeStruct((N, FEAT), jnp.float32),
            jax.ShapeDtypeStruct((2, FEAT), jnp.float32),
        ],
        scratch_shapes=[pltpu.VMEM((2, FEAT), jnp.float32)],
    )(p, dinv_col, b.reshape(1, FEAT))


def _edge_stage(xl, src3, he3, he, e, dinv_denom_col, bv_inv_col, d_inv_col,
                b, z_he, z_node):
    dinv_flat = dinv_denom_col.reshape(MP)
    alpha = _sc_alpha2(he, e, dinv_flat)
    hp = _sc_msg_he(xl, src3, he3, alpha, z_he)
    hout = _chout(hp, bv_inv_col)
    op = _sc_msg_node(hout, he3, src3, alpha, z_node)
    return _chraw(op, d_inv_col, b)


def kernel(x, edge_index, edge_attr, W1, att1, b1, gn1_w, gn1_b, gn1_ms,
           Wf1, bf1, W2, att2, b2, gn2_w, gn2_b, gn2_ms, Wf2, bf2,
           A1, bA1, A2, bA2, Wc, bc):
    src = edge_index[0]
    he = edge_index[1]
    src3 = src.reshape(NW, NB, K_E)
    he3 = he.reshape(NW, NB, K_E)
    z_he = jnp.zeros((MP // NS, FEAT), jnp.float32)
    z_node = jnp.zeros((N // NS, FEAT), jnp.float32)

    cnp, cmp_ = _sc_count(src, he)
    d_inv_col = _colsum(cnp, "inv").reshape(N, 1)
    bv_inv_col = _colsum(cmp_, "inv").reshape(MP, 1)

    xl1, a1 = _pre(x, W1, att1[:FEAT])
    eal1, b1he = _pre(edge_attr, W1, att1[FEAT:])
    eal2, b2he = _pre(edge_attr, W2, att2[FEAT:])
    a1 = a1.reshape(N)
    b1p = jnp.concatenate([b1he.reshape(M), jnp.zeros(MP - M, jnp.float32)])
    b2p = jnp.concatenate([b2he.reshape(M), jnp.zeros(MP - M, jnp.float32)])

    e1, dp1 = _sc_alpha(src, he, a1, b1p)
    dinv1 = _colsum(dp1, "rcp")
    h1raw, st1 = _edge_stage(xl1, src3, he3, he, e1, dinv1, bv_inv_col,
                             d_inv_col, b1, z_he, z_node)
    _, out1, xl2, a2 = _apply(h1raw, st1, gn1_w, gn1_b, gn1_ms, Wf1, bf1,
                              W2, att2[:FEAT])
    a2 = a2.reshape(N)

    e2, dp2 = _sc_alpha(src, he, a2, b2p)
    dinv2 = _colsum(dp2, "rcp")
    h2raw, st2 = _edge_stage(xl2, src3, he3, he, e2, dinv2, bv_inv_col,
                             d_inv_col, b2, z_he, z_node)
    _, out2, _, _ = _apply(h2raw, st2, gn2_w, gn2_b, gn2_ms, Wf2, bf2,
                           W2, att2[:FEAT])

    out = jnp.concatenate([x, out1, out2], axis=1)
    sc = _pool(out, A1, bA1, A2, bA2)
    sc_row = sc.reshape(1, 2 * FEAT)
    return _logits(out, sc_row, Wc, bc)
